# baseline (speedup 1.0000x reference)
import functools

import jax
import jax.numpy as jnp
from jax import lax
from jax.experimental import pallas as pl
from jax.experimental.pallas import tpu as pltpu
from jax.experimental.pallas import tpu_sc as plsc

ALPHA = 0.1
NC = 2
NS = 16
NW = NC * NS
EBLK = 112
IDX_CHUNK = 16
NBUF = 3

_mesh = functools.partial(
    plsc.VectorSubcoreMesh,
    core_axis_name="c", subcore_axis_name="s", num_cores=NC, num_subcores=NS,
)


def _spmm_sc(nb_per_worker, np_rows, d, n_src):
  rows_per_worker = np_rows // NS

  @functools.partial(
      pl.kernel,
      out_type=jax.ShapeDtypeStruct((NC, np_rows, d), jnp.float32),
      mesh=_mesh(),
      scratch_types=[
          pltpu.VMEM((IDX_CHUNK, EBLK), jnp.int32),
          pltpu.VMEM((IDX_CHUNK, EBLK), jnp.int32),
          pltpu.VMEM((NBUF, EBLK, d), jnp.float32),
          pltpu.VMEM_SHARED((np_rows, d), jnp.float32),
      ] + [pltpu.SemaphoreType.DMA] * (2 * NBUF),
  )
  def k(src_hbm, col_hbm, row_hbm, out_hbm, col_v, row_v, fbuf, acc,
        *sems):
    gsems = sems[:NBUF]
    ssems = sems[NBUF:]
    c = lax.axis_index("c")
    s = lax.axis_index("s")

    zbase = s * rows_per_worker
    zv = jnp.zeros((16,), jnp.float32)

    def zrow(i, _):
      for j in range(d // 16):
        fbuf[0, i, pl.ds(j * 16, 16)] = zv
      return 0

    lax.fori_loop(0, EBLK, zrow, 0)
    left = rows_per_worker
    off = 0
    while left > 0:
      step = min(EBLK, left)
      pltpu.sync_copy(fbuf.at[0, pl.ds(0, step)],
                      acc.at[pl.ds(zbase + off, step)])
      off += step
      left -= step
    plsc.subcore_barrier()

    wblk = (c * NS + s) * nb_per_worker

    def body(kk, _):
      base = wblk + kk * IDX_CHUNK
      pltpu.sync_copy(col_hbm.at[pl.ds(base, IDX_CHUNK)], col_v)
      pltpu.sync_copy(row_hbm.at[pl.ds(base, IDX_CHUNK)], row_v)
      gd = [pltpu.make_async_copy(src_hbm.at[col_v.at[j]], fbuf.at[j % NBUF],
                                  gsems[j % NBUF])
            for j in range(IDX_CHUNK)]
      sd = [pltpu.make_async_copy(fbuf.at[j % NBUF], acc.at[row_v.at[j]],
                                  ssems[j % NBUF])
            for j in range(IDX_CHUNK)]
      for j in range(IDX_CHUNK):
        if j >= NBUF:
          sd[j - NBUF].wait()
        gd[j].start()
        if j >= NBUF - 1:
          jj = j - (NBUF - 1)
          gd[jj].wait()
          sd[jj].start(add=True)
      for jj in range(IDX_CHUNK - NBUF + 1, IDX_CHUNK):
        gd[jj].wait()
        sd[jj].start(add=True)
      for jj in range(IDX_CHUNK - NBUF, IDX_CHUNK):
        sd[jj].wait()
      return 0

    lax.fori_loop(0, nb_per_worker // IDX_CHUNK, body, 0)
    plsc.subcore_barrier()

    pltpu.sync_copy(acc.at[pl.ds(zbase, rows_per_worker)],
                    out_hbm.at[c, pl.ds(zbase, rows_per_worker)])

  return k


def _deg_sc(nb_per_worker, np_rows, d):
  rows_per_worker = np_rows // NS
  ndsem = 4

  @functools.partial(
      pl.kernel,
      out_type=jax.ShapeDtypeStruct((NC, np_rows, d), jnp.float32),
      mesh=_mesh(),
      scratch_types=[
          pltpu.VMEM((IDX_CHUNK, EBLK), jnp.int32),
          pltpu.VMEM((EBLK, d), jnp.float32),
          pltpu.VMEM_SHARED((np_rows, d), jnp.float32),
      ] + [pltpu.SemaphoreType.DMA] * ndsem,
  )
  def k(col_hbm, out_hbm, col_v, obuf, acc, *sems):
    c = lax.axis_index("c")
    s = lax.axis_index("s")
    zbase = s * rows_per_worker

    zv = jnp.zeros((16,), jnp.float32)

    def zrow(i, _):
      for j in range(d // 16):
        obuf[i, pl.ds(j * 16, 16)] = zv
      return 0

    lax.fori_loop(0, EBLK, zrow, 0)
    left = rows_per_worker
    off = 0
    while left > 0:
      step = min(EBLK, left)
      pltpu.sync_copy(obuf.at[pl.ds(0, step)],
                      acc.at[pl.ds(zbase + off, step)])
      off += step
      left -= step

    ov = jnp.ones((16,), jnp.float32)

    def orow(i, _):
      for j in range(d // 16):
        obuf[i, pl.ds(j * 16, 16)] = ov
      return 0

    lax.fori_loop(0, EBLK, orow, 0)
    plsc.subcore_barrier()

    wblk = (c * NS + s) * nb_per_worker

    def body(kk, _):
      base = wblk + kk * IDX_CHUNK
      pltpu.sync_copy(col_hbm.at[pl.ds(base, IDX_CHUNK)], col_v)
      sd = [pltpu.make_async_copy(obuf, acc.at[col_v.at[j]], sems[j % ndsem])
            for j in range(IDX_CHUNK)]
      for j in range(IDX_CHUNK):
        if j >= ndsem:
          sd[j - ndsem].wait()
        sd[j].start(add=True)
      for j in range(IDX_CHUNK - ndsem, IDX_CHUNK):
        sd[j].wait()
      return 0

    lax.fori_loop(0, nb_per_worker // IDX_CHUNK, body, 0)
    plsc.subcore_barrier()

    pltpu.sync_copy(acc.at[pl.ds(zbase, rows_per_worker)],
                    out_hbm.at[c, pl.ds(zbase, rows_per_worker)])

  return k


_BR = 1000


def _tc_call(body, n, in_widths, out_widths, arrays):
  grid = n // _BR
  in_specs = [
      pl.BlockSpec((_BR, w), lambda i: (i, 0)) for w in in_widths
  ]
  out_specs = [
      pl.BlockSpec((_BR, w), lambda i: (i, 0)) for w in out_widths
  ]
  out_shape = [jax.ShapeDtypeStruct((n, w), jnp.float32) for w in out_widths]
  res = pl.pallas_call(
      body, grid=(grid,), in_specs=in_specs, out_specs=out_specs,
      out_shape=out_shape)(*arrays)
  return res


def _matmul_tc(x, w):
  n, d_in = x.shape
  d_out = w.shape[1]

  def body(x_ref, w_ref, o_ref):
    o_ref[...] = jnp.dot(x_ref[...], w_ref[...],
                         preferred_element_type=jnp.float32)

  grid = n // _BR
  return pl.pallas_call(
      body, grid=(grid,),
      in_specs=[pl.BlockSpec((_BR, d_in), lambda i: (i, 0)),
                pl.BlockSpec((d_in, d_out), lambda i: (0, 0))],
      out_specs=pl.BlockSpec((_BR, d_out), lambda i: (i, 0)),
      out_shape=jax.ShapeDtypeStruct((n, d_out), jnp.float32))(x, w)


def _vec_first_tc(dega, degb, s0):
  n, d = s0.shape

  def body(da_ref, db_ref, s0_ref, dis_ref, di_ref, sg_ref, ss_ref):
    deg = da_ref[:, 0:1] + db_ref[:, 0:1]
    degs = jnp.maximum(deg, 1e-12)
    pos = deg > 0
    dis = jnp.where(pos, lax.rsqrt(degs), 0.0)
    di = jnp.where(pos, 1.0 / degs, 0.0)
    dis_ref[...] = dis
    di_ref[...] = di
    s0v = s0_ref[...]
    sg_ref[...] = dis * s0v
    ss_ref[...] = di * s0v

  return _tc_call(body, n, [d, d, d], [1, 1, d, d], [dega, degb, s0])


def _gcn_post_tc(pa, pb, dis):
  n, d = pa.shape

  def body(pa_ref, pb_ref, dis_ref, g_ref, src_ref):
    g = dis_ref[...] * (pa_ref[...] + pb_ref[...])
    g_ref[...] = g
    src_ref[...] = dis_ref[...] * g

  return _tc_call(body, n, [d, d, 1], [d, d], [pa, pb, dis])


def _sct_post_tc(pa, pb, v, di):
  n, d = pa.shape

  def body(pa_ref, pb_ref, v_ref, di_ref, w_ref, src_ref):
    w = 0.5 * (v_ref[...] + pa_ref[...] + pb_ref[...])
    w_ref[...] = w
    src_ref[...] = di_ref[...] * w

  return _tc_call(body, n, [d, d, d, 1], [d, d], [pa, pb, v, di])


def _epilogue_tc(h, g1, g2, g3, p1, p2, p4, p8, a1, a2):
  n, d = h.shape

  def body(h_ref, g1_ref, g2_ref, g3_ref, p1_ref, p2_ref, p4_ref, p8_ref,
           a1_ref, a2_ref, hp_ref, att_ref):
    def lr(t):
      return jnp.where(t > 0, t, ALPHA * t)

    hv = h_ref[...]
    a1v = a1_ref[...]
    a2v = a2_ref[...]
    s1 = jnp.abs(p1_ref[...] - p2_ref[...])
    s2 = jnp.abs(p2_ref[...] - p4_ref[...])
    s3 = jnp.abs(p4_ref[...] - p8_ref[...])
    branches = (g1_ref[...], g2_ref[...], g3_ref[...], s1, s2, s3)
    ch = jnp.dot(lr(hv), a1v, preferred_element_type=jnp.float32)
    es = [ch + jnp.dot(lr(b), a2v, preferred_element_type=jnp.float32)
          for b in branches]
    e = jnp.concatenate(es, axis=1)
    m = jnp.max(e, axis=1, keepdims=True)
    p = jnp.exp(e - m)
    att = p / jnp.sum(p, axis=1, keepdims=True)
    att_ref[...] = att
    acc = att[:, 0:1] * branches[0]
    for k in range(1, 6):
      acc = acc + att[:, k:k + 1] * branches[k]
    hp_ref[...] = acc * (1.0 / 6.0)

  grid = n // _BR
  in_specs = [pl.BlockSpec((_BR, d), lambda i: (i, 0)) for _ in range(8)]
  in_specs += [pl.BlockSpec((d, 1), lambda i: (0, 0))] * 2
  out_specs = [pl.BlockSpec((_BR, d), lambda i: (i, 0)),
               pl.BlockSpec((_BR, 6), lambda i: (i, 0))]
  out_shape = [jax.ShapeDtypeStruct((n, d), jnp.float32),
               jax.ShapeDtypeStruct((n, 6), jnp.float32)]
  return pl.pallas_call(
      body, grid=(grid,), in_specs=in_specs, out_specs=out_specs,
      out_shape=out_shape)(h, g1, g2, g3, p1, p2, p4, p8, a1, a2)


def kernel(input, edge_index, W_mlp, a_weight):
  x = input
  n, d = x.shape
  e = edge_index.shape[1]

  nb = -(-e // (NW * EBLK * IDX_CHUNK)) * IDX_CHUNK
  ep = NW * nb * EBLK
  pad = ep - e
  np_rows = -(-(n + 1) // (8 * NS)) * (8 * NS)

  row = edge_index[0]
  col = edge_index[1]
  i32 = jnp.int32
  rowp = jnp.concatenate([row, jnp.full((pad,), n, i32)]).reshape(-1, EBLK)
  colg = jnp.concatenate([col, jnp.zeros((pad,), i32)]).reshape(-1, EBLK)
  cold = jnp.concatenate([col, jnp.full((pad,), n, i32)]).reshape(-1, EBLK)

  spmm = _spmm_sc(nb, np_rows, d, n)

  degp = _deg_sc(nb, np_rows, d)(cold)
  s0 = _matmul_tc(x, W_mlp)

  dis, di, src_g, src_s = _vec_first_tc(degp[0, :n], degp[1, :n], s0)

  gcn = []
  src = src_g
  for _ in range(3):
    pp = spmm(src, colg, rowp)
    g, src = _gcn_post_tc(pp[0, :n], pp[1, :n], dis)
    gcn.append(g)

  v = s0
  src = src_s
  saves = []
  for step in range(8):
    pp = spmm(src, colg, rowp)
    v, src = _sct_post_tc(pp[0, :n], pp[1, :n], v, di)
    if step in (0, 1, 3, 7):
      saves.append(v)
  p1, p2, p4, p8 = saves

  hp, att = _epilogue_tc(s0, gcn[0], gcn[1], gcn[2], p1, p2, p4, p8,
                         a_weight[:d], a_weight[d:])
  return hp, att.reshape(n, 6, 1)


# speedup vs baseline: 2.2638x; 2.2638x over previous
import functools

import jax
import jax.numpy as jnp
from jax import lax
from jax.experimental import pallas as pl
from jax.experimental.pallas import tpu as pltpu
from jax.experimental.pallas import tpu_sc as plsc

ALPHA = 0.1
NC = 2
NS = 16
NW = NC * NS
EBLK = 128
IDX_CHUNK = 16
NBUF = 2

_mesh = functools.partial(
    plsc.VectorSubcoreMesh,
    core_axis_name="c", subcore_axis_name="s", num_cores=NC, num_subcores=NS,
)


def _spmm_sc(nb_per_worker, np_rows, d, n_src):
  rows_per_worker = np_rows // NS

  @functools.partial(
      pl.kernel,
      out_type=jax.ShapeDtypeStruct((NC, np_rows, d), jnp.float32),
      mesh=_mesh(),
      scratch_types=[
          pltpu.VMEM((IDX_CHUNK, EBLK), jnp.int32),
          pltpu.VMEM((IDX_CHUNK, EBLK), jnp.int32),
          pltpu.VMEM((NBUF, EBLK, d), jnp.float32),
          pltpu.VMEM_SHARED((np_rows, d), jnp.float32),
      ] + [pltpu.SemaphoreType.DMA] * (2 * NBUF),
  )
  def k(src_hbm, col_hbm, row_hbm, out_hbm, col_v, row_v, fbuf, acc,
        *sems):
    gsems = sems[:NBUF]
    ssems = sems[NBUF:]
    c = lax.axis_index("c")
    s = lax.axis_index("s")

    zbase = s * rows_per_worker
    zv = jnp.zeros((16,), jnp.float32)

    def zrow(i, _):
      for j in range(d // 16):
        fbuf[0, i, pl.ds(j * 16, 16)] = zv
      return 0

    lax.fori_loop(0, EBLK, zrow, 0)
    left = rows_per_worker
    off = 0
    while left > 0:
      step = min(EBLK, left)
      pltpu.sync_copy(fbuf.at[0, pl.ds(0, step)],
                      acc.at[pl.ds(zbase + off, step)])
      off += step
      left -= step
    plsc.subcore_barrier()

    wblk = (c * NS + s) * nb_per_worker

    def body(kk, _):
      base = wblk + kk * IDX_CHUNK
      pltpu.sync_copy(col_hbm.at[pl.ds(base, IDX_CHUNK)], col_v)
      pltpu.sync_copy(row_hbm.at[pl.ds(base, IDX_CHUNK)], row_v)
      gd = [pltpu.make_async_copy(src_hbm.at[col_v.at[j]], fbuf.at[j % NBUF],
                                  gsems[j % NBUF])
            for j in range(IDX_CHUNK)]
      sd = [pltpu.make_async_copy(fbuf.at[j % NBUF], acc.at[row_v.at[j]],
                                  ssems[j % NBUF])
            for j in range(IDX_CHUNK)]
      for j in range(IDX_CHUNK):
        if j >= NBUF:
          sd[j - NBUF].wait()
        gd[j].start()
        if j >= NBUF - 1:
          jj = j - (NBUF - 1)
          gd[jj].wait()
          sd[jj].start(add=True)
      for jj in range(IDX_CHUNK - NBUF + 1, IDX_CHUNK):
        gd[jj].wait()
        sd[jj].start(add=True)
      for jj in range(IDX_CHUNK - NBUF, IDX_CHUNK):
        sd[jj].wait()
      return 0

    lax.fori_loop(0, nb_per_worker // IDX_CHUNK, body, 0)
    plsc.subcore_barrier()

    pltpu.sync_copy(acc.at[pl.ds(zbase, rows_per_worker)],
                    out_hbm.at[c, pl.ds(zbase, rows_per_worker)])

  return k


def _deg_sc(nb_per_worker, np_rows, d):
  rows_per_worker = np_rows // NS
  ndsem = 4

  @functools.partial(
      pl.kernel,
      out_type=jax.ShapeDtypeStruct((NC, np_rows, d), jnp.float32),
      mesh=_mesh(),
      scratch_types=[
          pltpu.VMEM((IDX_CHUNK, EBLK), jnp.int32),
          pltpu.VMEM((EBLK, d), jnp.float32),
          pltpu.VMEM_SHARED((np_rows, d), jnp.float32),
      ] + [pltpu.SemaphoreType.DMA] * ndsem,
  )
  def k(col_hbm, out_hbm, col_v, obuf, acc, *sems):
    c = lax.axis_index("c")
    s = lax.axis_index("s")
    zbase = s * rows_per_worker

    zv = jnp.zeros((16,), jnp.float32)

    def zrow(i, _):
      for j in range(d // 16):
        obuf[i, pl.ds(j * 16, 16)] = zv
      return 0

    lax.fori_loop(0, EBLK, zrow, 0)
    left = rows_per_worker
    off = 0
    while left > 0:
      step = min(EBLK, left)
      pltpu.sync_copy(obuf.at[pl.ds(0, step)],
                      acc.at[pl.ds(zbase + off, step)])
      off += step
      left -= step

    ov = jnp.ones((16,), jnp.float32)

    def orow(i, _):
      for j in range(d // 16):
        obuf[i, pl.ds(j * 16, 16)] = ov
      return 0

    lax.fori_loop(0, EBLK, orow, 0)
    plsc.subcore_barrier()

    wblk = (c * NS + s) * nb_per_worker

    def body(kk, _):
      base = wblk + kk * IDX_CHUNK
      pltpu.sync_copy(col_hbm.at[pl.ds(base, IDX_CHUNK)], col_v)
      sd = [pltpu.make_async_copy(obuf, acc.at[col_v.at[j]], sems[j % ndsem])
            for j in range(IDX_CHUNK)]
      for j in range(IDX_CHUNK):
        if j >= ndsem:
          sd[j - ndsem].wait()
        sd[j].start(add=True)
      for j in range(IDX_CHUNK - ndsem, IDX_CHUNK):
        sd[j].wait()
      return 0

    lax.fori_loop(0, nb_per_worker // IDX_CHUNK, body, 0)
    plsc.subcore_barrier()

    pltpu.sync_copy(acc.at[pl.ds(zbase, rows_per_worker)],
                    out_hbm.at[c, pl.ds(zbase, rows_per_worker)])

  return k


_BR = 1000


def _tc_call(body, n, in_widths, out_widths, arrays):
  grid = n // _BR
  in_specs = [
      pl.BlockSpec((_BR, w), lambda i: (i, 0)) for w in in_widths
  ]
  out_specs = [
      pl.BlockSpec((_BR, w), lambda i: (i, 0)) for w in out_widths
  ]
  out_shape = [jax.ShapeDtypeStruct((n, w), jnp.float32) for w in out_widths]
  res = pl.pallas_call(
      body, grid=(grid,), in_specs=in_specs, out_specs=out_specs,
      out_shape=out_shape)(*arrays)
  return res


def _matmul_tc(x, w):
  n, d_in = x.shape
  d_out = w.shape[1]

  def body(x_ref, w_ref, o_ref):
    o_ref[...] = jnp.dot(x_ref[...], w_ref[...],
                         preferred_element_type=jnp.float32)

  grid = n // _BR
  return pl.pallas_call(
      body, grid=(grid,),
      in_specs=[pl.BlockSpec((_BR, d_in), lambda i: (i, 0)),
                pl.BlockSpec((d_in, d_out), lambda i: (0, 0))],
      out_specs=pl.BlockSpec((_BR, d_out), lambda i: (i, 0)),
      out_shape=jax.ShapeDtypeStruct((n, d_out), jnp.float32))(x, w)


def _vec_first_tc(dega, degb, s0):
  n, d = s0.shape

  def body(da_ref, db_ref, s0_ref, dis_ref, di_ref, sg_ref, ss_ref):
    deg = da_ref[:, 0:1] + db_ref[:, 0:1]
    degs = jnp.maximum(deg, 1e-12)
    pos = deg > 0
    dis = jnp.where(pos, lax.rsqrt(degs), 0.0)
    di = jnp.where(pos, 1.0 / degs, 0.0)
    dis_ref[...] = dis
    di_ref[...] = di
    s0v = s0_ref[...]
    sg_ref[...] = dis * s0v
    ss_ref[...] = di * s0v

  return _tc_call(body, n, [d, d, d], [1, 1, d, d], [dega, degb, s0])


def _gcn_post_tc(pa, pb, dis):
  n, d = pa.shape

  def body(pa_ref, pb_ref, dis_ref, g_ref, src_ref):
    g = dis_ref[...] * (pa_ref[...] + pb_ref[...])
    g_ref[...] = g
    src_ref[...] = dis_ref[...] * g

  return _tc_call(body, n, [d, d, 1], [d, d], [pa, pb, dis])


def _sct_post_tc(pa, pb, v, di):
  n, d = pa.shape

  def body(pa_ref, pb_ref, v_ref, di_ref, w_ref, src_ref):
    w = 0.5 * (v_ref[...] + pa_ref[...] + pb_ref[...])
    w_ref[...] = w
    src_ref[...] = di_ref[...] * w

  return _tc_call(body, n, [d, d, d, 1], [d, d], [pa, pb, v, di])


def _epilogue_tc(h, g1, g2, g3, p1, p2, p4, p8, a1, a2):
  n, d = h.shape

  def body(h_ref, g1_ref, g2_ref, g3_ref, p1_ref, p2_ref, p4_ref, p8_ref,
           a1_ref, a2_ref, hp_ref, att_ref):
    def lr(t):
      return jnp.where(t > 0, t, ALPHA * t)

    hv = h_ref[...]
    a1v = a1_ref[...]
    a2v = a2_ref[...]
    s1 = jnp.abs(p1_ref[...] - p2_ref[...])
    s2 = jnp.abs(p2_ref[...] - p4_ref[...])
    s3 = jnp.abs(p4_ref[...] - p8_ref[...])
    branches = (g1_ref[...], g2_ref[...], g3_ref[...], s1, s2, s3)
    ch = jnp.dot(lr(hv), a1v, preferred_element_type=jnp.float32)
    es = [ch + jnp.dot(lr(b), a2v, preferred_element_type=jnp.float32)
          for b in branches]
    e = jnp.concatenate(es, axis=1)
    m = jnp.max(e, axis=1, keepdims=True)
    p = jnp.exp(e - m)
    att = p / jnp.sum(p, axis=1, keepdims=True)
    att_ref[...] = att
    acc = att[:, 0:1] * branches[0]
    for k in range(1, 6):
      acc = acc + att[:, k:k + 1] * branches[k]
    hp_ref[...] = acc * (1.0 / 6.0)

  grid = n // _BR
  in_specs = [pl.BlockSpec((_BR, d), lambda i: (i, 0)) for _ in range(8)]
  in_specs += [pl.BlockSpec((d, 1), lambda i: (0, 0))] * 2
  out_specs = [pl.BlockSpec((_BR, d), lambda i: (i, 0)),
               pl.BlockSpec((_BR, 6), lambda i: (i, 0))]
  out_shape = [jax.ShapeDtypeStruct((n, d), jnp.float32),
               jax.ShapeDtypeStruct((n, 6), jnp.float32)]
  return pl.pallas_call(
      body, grid=(grid,), in_specs=in_specs, out_specs=out_specs,
      out_shape=out_shape)(h, g1, g2, g3, p1, p2, p4, p8, a1, a2)


def kernel(input, edge_index, W_mlp, a_weight):
  x = input
  n, d = x.shape
  e = edge_index.shape[1]

  nb = -(-e // (NW * EBLK * IDX_CHUNK)) * IDX_CHUNK
  ep = NW * nb * EBLK
  pad = ep - e
  np_rows = -(-(n + 1) // (8 * NS)) * (8 * NS)

  row = edge_index[0]
  col = edge_index[1]
  i32 = jnp.int32
  rowp = jnp.concatenate([row, jnp.full((pad,), n, i32)]).reshape(-1, EBLK)
  colg = jnp.concatenate([col, jnp.zeros((pad,), i32)]).reshape(-1, EBLK)
  cold = jnp.concatenate([col, jnp.full((pad,), n, i32)]).reshape(-1, EBLK)

  spmm = _spmm_sc(nb, np_rows, d, n)

  degp = _deg_sc(nb, np_rows, d)(cold)
  s0 = _matmul_tc(x, W_mlp)

  dis, di, src_g, src_s = _vec_first_tc(degp[0, :n], degp[1, :n], s0)

  gcn = []
  src = src_g
  for _ in range(3):
    pp = spmm(src, colg, rowp)
    g, src = _gcn_post_tc(pp[0, :n], pp[1, :n], dis)
    gcn.append(g)

  v = s0
  src = src_s
  saves = []
  for step in range(8):
    pp = spmm(src, colg, rowp)
    v, src = _sct_post_tc(pp[0, :n], pp[1, :n], v, di)
    if step in (0, 1, 3, 7):
      saves.append(v)
  p1, p2, p4, p8 = saves

  hp, att = _epilogue_tc(s0, gcn[0], gcn[1], gcn[2], p1, p2, p4, p8,
                         a_weight[:d], a_weight[d:])
  return hp, att.reshape(n, 6, 1)
